# baseline (speedup 1.0000x reference)
import functools

import jax
import jax.numpy as jnp
from jax import lax
from jax.experimental import pallas as pl
from jax.experimental.pallas import tpu as pltpu
from jax.experimental.pallas import tpu_sc as plsc

BATCH = 16384
EMB = 32
NC = 2
NS = 16
NW = NC * NS
BW = BATCH // NW
PACK = 128 // EMB


def _sc_gather_body(uid_hbm, iid_hbm, uemb_hbm, iemb_hbm, ubias_hbm,
                    ibias_hbm, eu_out, ev_out, ub_out, ib_out,
                    uidx_v, iidx_v, eu_v, ev_v, ub_v, ib_v,
                    sem_eu, sem_ev, sem_ub, sem_ib, sem_w):
    wid = lax.axis_index("s") * NC + lax.axis_index("c")
    base = wid * BW
    sl = pl.ds(base, BW)
    pltpu.sync_copy(uid_hbm.at[sl], uidx_v)
    pltpu.sync_copy(iid_hbm.at[sl], iidx_v)
    cp_eu = pltpu.async_copy(uemb_hbm.at[uidx_v], eu_v, sem_eu)
    cp_ev = pltpu.async_copy(iemb_hbm.at[iidx_v], ev_v, sem_ev)
    cp_ub = pltpu.async_copy(ubias_hbm.at[uidx_v], ub_v, sem_ub)
    cp_ib = pltpu.async_copy(ibias_hbm.at[iidx_v], ib_v, sem_ib)
    cp_eu.wait()
    cp_ev.wait()
    cp_ub.wait()
    cp_ib.wait()
    cp0 = pltpu.async_copy(eu_v, eu_out.at[sl], sem_w)
    cp1 = pltpu.async_copy(ev_v, ev_out.at[sl], sem_w)
    cp2 = pltpu.async_copy(ub_v, ub_out.at[sl], sem_w)
    cp3 = pltpu.async_copy(ib_v, ib_out.at[sl], sem_w)
    cp0.wait()
    cp1.wait()
    cp2.wait()
    cp3.wait()


@jax.jit
def _sc_gather(uid, iid, user_emb, item_emb, ubias, ibias):
    mesh = plsc.VectorSubcoreMesh(core_axis_name="c", subcore_axis_name="s",
                                  num_cores=NC, num_subcores=NS)
    return pl.kernel(
        _sc_gather_body,
        out_type=(
            jax.ShapeDtypeStruct((BATCH, EMB), jnp.float32),
            jax.ShapeDtypeStruct((BATCH, EMB), jnp.float32),
            jax.ShapeDtypeStruct((BATCH, 1), jnp.float32),
            jax.ShapeDtypeStruct((BATCH, 1), jnp.float32),
        ),
        mesh=mesh,
        scratch_types=[
            pltpu.VMEM((BW,), jnp.int32),
            pltpu.VMEM((BW,), jnp.int32),
            pltpu.VMEM((BW, EMB), jnp.float32),
            pltpu.VMEM((BW, EMB), jnp.float32),
            pltpu.VMEM((BW, 1), jnp.float32),
            pltpu.VMEM((BW, 1), jnp.float32),
            pltpu.SemaphoreType.DMA,
            pltpu.SemaphoreType.DMA,
            pltpu.SemaphoreType.DMA,
            pltpu.SemaphoreType.DMA,
            pltpu.SemaphoreType.DMA,
        ],
        compiler_params=pltpu.CompilerParams(use_tc_tiling_on_sc=False),
        name="recsys_sc_gather",
    )(uid, iid, user_emb, item_emb, ubias, ibias)


def _block_diag4(w):
    tiled = jnp.concatenate([jnp.concatenate([w] * PACK, axis=1)] * PACK,
                            axis=0)
    r = lax.broadcasted_iota(jnp.int32, (PACK * EMB, PACK * EMB), 0)
    c = lax.broadcasted_iota(jnp.int32, (PACK * EMB, PACK * EMB), 1)
    return jnp.where((r // EMB) == (c // EMB), tiled, jnp.float32(0))


def _tc_dense_body(eu_ref, ev_ref, uw_ref, ub_ref, iw_ref, ib_ref,
                   ubias_ref, ibias_ref, out_ref):
    f32 = jnp.float32
    wd_u = _block_diag4(uw_ref[...])
    wd_i = _block_diag4(iw_ref[...])
    b_u = jnp.concatenate([ub_ref[...]] * PACK, axis=0)
    b_i = jnp.concatenate([ib_ref[...]] * PACK, axis=0)
    eu = eu_ref[...]
    ev = ev_ref[...]
    u1 = jnp.dot(eu, wd_u, preferred_element_type=f32) + b_u
    u2 = jnp.dot(u1, wd_u, preferred_element_type=f32) + b_u
    v1 = jnp.dot(ev, wd_i, preferred_element_type=f32) + b_i
    v2 = jnp.dot(v1, wd_i, preferred_element_type=f32) + b_i
    s = jnp.sum(u2 * v2)
    x = s + ubias_ref[...] + ibias_ref[...]
    out_ref[...] = jax.nn.sigmoid(x)


@jax.jit
def _tc_dense(eu, ev, user_W, user_b, item_W, item_b, ubias, ibias):
    vmem = functools.partial(pl.BlockSpec, memory_space=pltpu.VMEM)
    return pl.pallas_call(
        _tc_dense_body,
        out_shape=jax.ShapeDtypeStruct((BATCH, 1), jnp.float32),
        in_specs=[vmem()] * 8,
        out_specs=vmem(),
        name="recsys_tc_dense",
    )(eu, ev, user_W, user_b, item_W, item_b, ubias, ibias)


def kernel(inputs, user_emb, user_W, user_b, user_bias_tab, item_emb,
           item_W, item_b, item_bias_tab):
    uid = inputs[:, 0]
    iid = inputs[:, 1]
    eu, ev, ub, ib = _sc_gather(uid, iid, user_emb, item_emb,
                                user_bias_tab, item_bias_tab)
    return _tc_dense(eu.reshape(BATCH // PACK, 128),
                     ev.reshape(BATCH // PACK, 128),
                     user_W, user_b, item_W, item_b, ub, ib)


# speedup vs baseline: 2.8928x; 2.8928x over previous
import functools

import jax
import jax.numpy as jnp
from jax import lax
from jax.experimental import pallas as pl
from jax.experimental.pallas import tpu as pltpu
from jax.experimental.pallas import tpu_sc as plsc

BATCH = 16384
EMB = 32
NC = 2
NS = 16
NW = NC * NS
BW = BATCH // NW
PACK = 128 // EMB


def _sc_gather_body(uid_hbm, iid_hbm, uemb_hbm, iemb_hbm, ubias_hbm,
                    ibias_hbm, eu_out, ev_out, ub_out, ib_out,
                    uidx_v, iidx_v, eu_v, ev_v, ub_v, ib_v,
                    sem_eu, sem_ev, sem_ub, sem_ib, sem_w):
    wid = lax.axis_index("s") * NC + lax.axis_index("c")
    base = wid * BW
    sl = pl.ds(base, BW)
    pltpu.sync_copy(uid_hbm.at[sl], uidx_v)
    pltpu.sync_copy(iid_hbm.at[sl], iidx_v)
    cp_eu = pltpu.async_copy(uemb_hbm.at[uidx_v], eu_v, sem_eu)
    cp_ev = pltpu.async_copy(iemb_hbm.at[iidx_v], ev_v, sem_ev)
    cp_ub = pltpu.async_copy(ubias_hbm.at[uidx_v], ub_v, sem_ub)
    cp_ib = pltpu.async_copy(ibias_hbm.at[iidx_v], ib_v, sem_ib)
    cp_eu.wait()
    cp_ev.wait()
    cp_ub.wait()
    cp_ib.wait()
    cp0 = pltpu.async_copy(eu_v, eu_out.at[sl], sem_w)
    cp1 = pltpu.async_copy(ev_v, ev_out.at[sl], sem_w)
    cp2 = pltpu.async_copy(ub_v, ub_out.at[sl], sem_w)
    cp3 = pltpu.async_copy(ib_v, ib_out.at[sl], sem_w)
    cp0.wait()
    cp1.wait()
    cp2.wait()
    cp3.wait()


@jax.jit
def _sc_gather(uid, iid, user_emb, item_emb, ubias, ibias):
    mesh = plsc.VectorSubcoreMesh(core_axis_name="c", subcore_axis_name="s",
                                  num_cores=NC, num_subcores=NS)
    return pl.kernel(
        _sc_gather_body,
        out_type=(
            jax.ShapeDtypeStruct((BATCH, EMB), jnp.float32),
            jax.ShapeDtypeStruct((BATCH, EMB), jnp.float32),
            jax.ShapeDtypeStruct((BATCH,), jnp.float32),
            jax.ShapeDtypeStruct((BATCH,), jnp.float32),
        ),
        mesh=mesh,
        scratch_types=[
            pltpu.VMEM((BW,), jnp.int32),
            pltpu.VMEM((BW,), jnp.int32),
            pltpu.VMEM((BW, EMB), jnp.float32),
            pltpu.VMEM((BW, EMB), jnp.float32),
            pltpu.VMEM((BW,), jnp.float32),
            pltpu.VMEM((BW,), jnp.float32),
            pltpu.SemaphoreType.DMA,
            pltpu.SemaphoreType.DMA,
            pltpu.SemaphoreType.DMA,
            pltpu.SemaphoreType.DMA,
            pltpu.SemaphoreType.DMA,
        ],
        compiler_params=pltpu.CompilerParams(use_tc_tiling_on_sc=False),
        name="recsys_sc_gather",
    )(uid, iid, user_emb, item_emb, ubias, ibias)


def _block_diag4(w):
    tiled = jnp.concatenate([jnp.concatenate([w] * PACK, axis=1)] * PACK,
                            axis=0)
    r = lax.broadcasted_iota(jnp.int32, (PACK * EMB, PACK * EMB), 0)
    c = lax.broadcasted_iota(jnp.int32, (PACK * EMB, PACK * EMB), 1)
    return jnp.where((r // EMB) == (c // EMB), tiled, jnp.float32(0))


def _tc_dense_body(eu_ref, ev_ref, uw_ref, ub_ref, iw_ref, ib_ref,
                   ubias_ref, ibias_ref, out_ref):
    f32 = jnp.float32
    wd_u = _block_diag4(uw_ref[...])
    wd_i = _block_diag4(iw_ref[...])
    b_u = jnp.concatenate([ub_ref[...]] * PACK, axis=0)
    b_i = jnp.concatenate([ib_ref[...]] * PACK, axis=0)
    eu = eu_ref[...]
    ev = ev_ref[...]
    u1 = jnp.dot(eu, wd_u, preferred_element_type=f32) + b_u
    u2 = jnp.dot(u1, wd_u, preferred_element_type=f32) + b_u
    v1 = jnp.dot(ev, wd_i, preferred_element_type=f32) + b_i
    v2 = jnp.dot(v1, wd_i, preferred_element_type=f32) + b_i
    s = jnp.sum(u2 * v2)
    x = s + ubias_ref[...] + ibias_ref[...]
    out_ref[...] = jax.nn.sigmoid(x)


@jax.jit
def _tc_dense(eu, ev, user_W, user_b, item_W, item_b, ubias, ibias):
    vmem = functools.partial(pl.BlockSpec, memory_space=pltpu.VMEM)
    return pl.pallas_call(
        _tc_dense_body,
        out_shape=jax.ShapeDtypeStruct((128, 128), jnp.float32),
        in_specs=[vmem()] * 8,
        out_specs=vmem(),
        name="recsys_tc_dense",
    )(eu, ev, user_W, user_b, item_W, item_b, ubias, ibias)


def kernel(inputs, user_emb, user_W, user_b, user_bias_tab, item_emb,
           item_W, item_b, item_bias_tab):
    uid = inputs[:, 0]
    iid = inputs[:, 1]
    eu, ev, ub, ib = _sc_gather(uid, iid, user_emb, item_emb,
                                user_bias_tab.T.reshape(-1),
                                item_bias_tab.T.reshape(-1))
    out = _tc_dense(eu.reshape(BATCH // PACK, 128),
                    ev.reshape(BATCH // PACK, 128),
                    user_W, user_b, item_W, item_b,
                    ub.reshape(128, 128), ib.reshape(128, 128))
    return out.reshape(BATCH, 1)


# speedup vs baseline: 3.8266x; 1.3228x over previous
import functools

import jax
import jax.numpy as jnp
from jax import lax
from jax.experimental import pallas as pl
from jax.experimental.pallas import tpu as pltpu
from jax.experimental.pallas import tpu_sc as plsc

BATCH = 16384
EMB = 32
NC = 2
NS = 16
NW = NC * NS
BW = BATCH // NW
CH = 128
NCH = BW // CH

_MESH = dict(core_axis_name="c", subcore_axis_name="s",
             num_cores=NC, num_subcores=NS)


def _sc_bias_body(uid_hbm, iid_hbm, ubT_hbm, ibT_hbm, ub_out, ib_out,
                  uidx_v, iidx_v, ub_v, ib_v, sem_u, sem_i, sem_w):
    wid = lax.axis_index("s") * NC + lax.axis_index("c")
    base = wid * BW
    sl = pl.ds(base, BW)
    pltpu.sync_copy(uid_hbm.at[sl], uidx_v)
    pltpu.sync_copy(iid_hbm.at[sl], iidx_v)
    cp_u = pltpu.async_copy(ubT_hbm.at[0].at[uidx_v], ub_v, sem_u)
    cp_i = pltpu.async_copy(ibT_hbm.at[0].at[iidx_v], ib_v, sem_i)
    cp_u.wait()
    cp_i.wait()
    cp0 = pltpu.async_copy(ub_v, ub_out.at[sl], sem_w)
    cp1 = pltpu.async_copy(ib_v, ib_out.at[sl], sem_w)
    cp0.wait()
    cp1.wait()


@jax.jit
def _sc_bias_gather(uid, iid, ubT, ibT):
    return pl.kernel(
        _sc_bias_body,
        out_type=(
            jax.ShapeDtypeStruct((BATCH,), jnp.float32),
            jax.ShapeDtypeStruct((BATCH,), jnp.float32),
        ),
        mesh=plsc.VectorSubcoreMesh(**_MESH),
        scratch_types=[
            pltpu.VMEM((BW,), jnp.int32),
            pltpu.VMEM((BW,), jnp.int32),
            pltpu.VMEM((BW,), jnp.float32),
            pltpu.VMEM((BW,), jnp.float32),
            pltpu.SemaphoreType.DMA,
            pltpu.SemaphoreType.DMA,
            pltpu.SemaphoreType.DMA,
        ],
        compiler_params=pltpu.CompilerParams(use_tc_tiling_on_sc=False),
        name="recsys_sc_bias",
    )(uid, iid, ubT, ibT)


def _sc_emb_body(uid_hbm, iid_hbm, uemb_hbm, iemb_hbm, eu_out, ev_out,
                 uidx_v, iidx_v, eu_v, ev_v, sem_eu, sem_ev, sem_w):
    wid = lax.axis_index("s") * NC + lax.axis_index("c")
    base = wid * BW
    pltpu.sync_copy(uid_hbm.at[pl.ds(base, BW)], uidx_v)
    pltpu.sync_copy(iid_hbm.at[pl.ds(base, BW)], iidx_v)

    def chunk(c, _):
        off = c * CH

        def fetch(g, _):
            uvec = uidx_v[pl.ds(off + g * 16, 16)]
            ivec = iidx_v[pl.ds(off + g * 16, 16)]
            for j in range(16):
                k = g * 16 + j
                pltpu.async_copy(uemb_hbm.at[pl.ds(uvec[j], 1)],
                                 eu_v.at[pl.ds(k, 1)], sem_eu)
                pltpu.async_copy(iemb_hbm.at[pl.ds(ivec[j], 1)],
                                 ev_v.at[pl.ds(k, 1)], sem_ev)
            return ()

        lax.fori_loop(0, CH // 16, fetch, ())

        def drain(k, _):
            pltpu.make_async_copy(uemb_hbm.at[pl.ds(0, 1)],
                                  eu_v.at[pl.ds(0, 1)], sem_eu).wait()
            pltpu.make_async_copy(iemb_hbm.at[pl.ds(0, 1)],
                                  ev_v.at[pl.ds(0, 1)], sem_ev).wait()
            return ()

        lax.fori_loop(0, CH, drain, (), unroll=8)

        row0 = base + off
        cp0 = pltpu.async_copy(eu_v, eu_out.at[pl.ds(row0, CH)], sem_w)
        cp1 = pltpu.async_copy(ev_v, ev_out.at[pl.ds(row0, CH)], sem_w)
        cp0.wait()
        cp1.wait()
        return ()

    lax.fori_loop(0, NCH, chunk, ())


@jax.jit
def _sc_emb_gather(uid, iid, user_emb, item_emb):
    return pl.kernel(
        _sc_emb_body,
        out_type=(
            jax.ShapeDtypeStruct((BATCH, EMB), jnp.float32),
            jax.ShapeDtypeStruct((BATCH, EMB), jnp.float32),
        ),
        mesh=plsc.VectorSubcoreMesh(**_MESH),
        scratch_types=[
            pltpu.VMEM((BW,), jnp.int32),
            pltpu.VMEM((BW,), jnp.int32),
            pltpu.VMEM((CH, EMB), jnp.float32),
            pltpu.VMEM((CH, EMB), jnp.float32),
            pltpu.SemaphoreType.DMA,
            pltpu.SemaphoreType.DMA,
            pltpu.SemaphoreType.DMA,
        ],
        compiler_params=pltpu.CompilerParams(needs_layout_passes=False),
        name="recsys_sc_emb",
    )(uid, iid, user_emb, item_emb)


def _tc_dense_body(eu_ref, ev_ref, uw_ref, ub_ref, iw_ref, ib_ref,
                   ubias_ref, ibias_ref, out_ref):
    f32 = jnp.float32
    uw = uw_ref[...]
    iw = iw_ref[...]
    ub = ub_ref[...]
    ib = ib_ref[...]
    eu = eu_ref[...]
    ev = ev_ref[...]
    u1 = jnp.dot(eu, uw, preferred_element_type=f32) + ub
    u2 = jnp.dot(u1, uw, preferred_element_type=f32) + ub
    v1 = jnp.dot(ev, iw, preferred_element_type=f32) + ib
    v2 = jnp.dot(v1, iw, preferred_element_type=f32) + ib
    s = jnp.sum(u2 * v2)
    x = s + ubias_ref[...] + ibias_ref[...]
    out_ref[...] = jax.nn.sigmoid(x)


@jax.jit
def _tc_dense(eu, ev, user_W, user_b, item_W, item_b, ubias, ibias):
    vmem = functools.partial(pl.BlockSpec, memory_space=pltpu.VMEM)
    return pl.pallas_call(
        _tc_dense_body,
        out_shape=jax.ShapeDtypeStruct((128, 128), jnp.float32),
        in_specs=[vmem()] * 8,
        out_specs=vmem(),
        name="recsys_tc_dense",
    )(eu, ev, user_W, user_b, item_W, item_b,
      ubias.reshape(128, 128), ibias.reshape(128, 128))


def kernel(inputs, user_emb, user_W, user_b, user_bias_tab, item_emb,
           item_W, item_b, item_bias_tab):
    uid = inputs[:, 0]
    iid = inputs[:, 1]
    ub, ib = _sc_bias_gather(uid, iid, user_bias_tab.T, item_bias_tab.T)
    eu, ev = _sc_emb_gather(uid, iid, user_emb, item_emb)
    out = _tc_dense(eu, ev, user_W, user_b, item_W, item_b, ub, ib)
    return out.reshape(BATCH, 1)
